# baseline (speedup 1.0000x reference)
import jax
import jax.numpy as jnp
import numpy as np
from jax.experimental import pallas as pl

_N_CLASSES = 64
_SELECTED_MAPS = [[3, 17, 42], [5, 9, 28, 51], [0, 12, 33], [7, 21, 44, 60], [2, 14, 39], [8, 26, 55, 63]]
_MIS_VAL = 0.5
_TB = 2048


def _mm_flat():
    m = np.zeros((8, _N_CLASSES), dtype=np.float32)
    for i, vals in enumerate(_SELECTED_MAPS):
        m[1 + i, vals] = 1.0
    mm = m.reshape(1, 512)
    mid = np.zeros((1, 512), dtype=np.float32)
    mid[0, 64:448] = 1.0
    return jnp.asarray(np.concatenate([mm, (_MIS_VAL - 1.0 - mm) * mid, mid], axis=0))


def _g_mat():
    g = np.zeros((512, 8), dtype=np.float32)
    for l in range(512):
        g[l, l >> 6] = 1.0
    return jnp.asarray(g)


def _loss_kernel(x_ref, prim_ref, a2s_ref, match_ref, mm_ref, g_ref, out_ref):
    x = x_ref[...]
    prim = prim_ref[...]
    a2s = a2s_ref[...]
    match = match_ref[...]

    lane = jax.lax.broadcasted_iota(jnp.int32, x.shape, 1)
    b0 = (lane & 64) != 0
    b1 = (lane & 128) != 0
    b2 = (lane & 256) != 0

    def sel(col_a, col_b, bit):
        return jnp.where(bit, col_b, col_a)

    p01 = sel(prim[:, 0:1], prim[:, 1:2], b0)
    p23 = sel(prim[:, 2:3], prim[:, 3:4], b0)
    p45 = sel(prim[:, 4:5], prim[:, 5:6], b0)
    p67 = sel(prim[:, 6:7], prim[:, 7:8], b0)
    p0123 = sel(p01, p23, b1)
    p4567 = sel(p45, p67, b1)
    prim_lane = sel(p0123, p4567, b2)

    c = lane & 63
    eq_p = (c == prim_lane).astype(jnp.float32)
    eq_a2 = (lane == a2s).astype(jnp.float32)

    mm = mm_ref[0:1, :]
    mmc = mm_ref[1:2, :]
    nonmatchf = (match == 0).astype(jnp.float32)
    tcoef = eq_p + eq_a2 + nonmatchf * (mm + eq_p * mmc)

    sp = jnp.maximum(x, 0.0) + jnp.log1p(jnp.exp(-jnp.abs(x)))
    y = sp - x * tcoef
    out_ref[...] = jnp.dot(y, g_ref[...], preferred_element_type=jnp.float32)


def kernel(logits, b_train_phrase, b_attitude_1, b_attitude_2, b_compare, b_matching):
    B = logits.shape[0]
    xf = logits.reshape(B, 512)
    primary = jnp.concatenate(
        [b_attitude_1, b_compare, b_train_phrase[:, -1:]], axis=1
    ).astype(jnp.int32)
    a2 = b_attitude_2.astype(jnp.int32)
    a2s = jnp.where(a2 >= _N_CLASSES, 1024, a2)
    match = b_matching.astype(jnp.int32)

    grid = (B // _TB,)
    return pl.pallas_call(
        _loss_kernel,
        grid=grid,
        in_specs=[
            pl.BlockSpec((_TB, 512), lambda i: (i, 0)),
            pl.BlockSpec((_TB, 8), lambda i: (i, 0)),
            pl.BlockSpec((_TB, 1), lambda i: (i, 0)),
            pl.BlockSpec((_TB, 1), lambda i: (i, 0)),
            pl.BlockSpec((3, 512), lambda i: (0, 0)),
            pl.BlockSpec((512, 8), lambda i: (0, 0)),
        ],
        out_specs=pl.BlockSpec((_TB, 8), lambda i: (i, 0)),
        out_shape=jax.ShapeDtypeStruct((B, 8), jnp.float32),
    )(xf, primary, a2s, match, _mm_flat(), _g_mat())


# speedup vs baseline: 1.0302x; 1.0302x over previous
import jax
import jax.numpy as jnp
import numpy as np
from jax.experimental import pallas as pl

_N_CLASSES = 64
_SELECTED_MAPS = [[3, 17, 42], [5, 9, 28, 51], [0, 12, 33], [7, 21, 44, 60], [2, 14, 39], [8, 26, 55, 63]]
_MIS_VAL = 0.5
_TB = 2048


def _mm_flat():
    m = np.zeros((8, _N_CLASSES), dtype=np.float32)
    for i, vals in enumerate(_SELECTED_MAPS):
        m[1 + i, vals] = 1.0
    mm = m.reshape(1, 512)
    mid = np.zeros((1, 512), dtype=np.float32)
    mid[0, 64:448] = 1.0
    return jnp.asarray(np.concatenate([mm, (_MIS_VAL - 1.0 - mm) * mid, mid], axis=0))


def _g_mat():
    g = np.zeros((512, 8), dtype=np.float32)
    for l in range(512):
        g[l, l >> 6] = 1.0
    return jnp.asarray(g)


def _loss_kernel(x_ref, prim_ref, a2s_ref, match_ref, mm_ref, g_ref, out_ref):
    x = x_ref[...]
    prim = prim_ref[...]
    a2s = a2s_ref[...]
    match = match_ref[...]

    lane = jax.lax.broadcasted_iota(jnp.int32, x.shape, 1)
    b0 = (lane & 64) != 0
    b1 = (lane & 128) != 0
    b2 = (lane & 256) != 0

    def sel(col_a, col_b, bit):
        return jnp.where(bit, col_b, col_a)

    p01 = sel(prim[:, 0:1], prim[:, 1:2], b0)
    p23 = sel(prim[:, 2:3], prim[:, 3:4], b0)
    p45 = sel(prim[:, 4:5], prim[:, 5:6], b0)
    p67 = sel(prim[:, 6:7], prim[:, 7:8], b0)
    p0123 = sel(p01, p23, b1)
    p4567 = sel(p45, p67, b1)
    prim_lane = sel(p0123, p4567, b2)

    c = lane & 63
    eq_p = (c == prim_lane).astype(jnp.float32)
    eq_a2 = (lane == a2s).astype(jnp.float32)

    mm = mm_ref[0:1, :]
    mmc = mm_ref[1:2, :]
    nonmatchf = (match == 0).astype(jnp.float32)
    tcoef = eq_p + eq_a2 + nonmatchf * (mm + eq_p * mmc)

    t = jnp.exp(-jnp.abs(x))
    log1p_t = ((0.1077468561780622 * t - 0.39711829644996904) * t
               + 0.9823971197982739) * t + 0.0005027216331518523
    sp = jnp.maximum(x, 0.0) + log1p_t
    y = sp - x * tcoef
    out_ref[...] = jnp.dot(y, g_ref[...], preferred_element_type=jnp.float32)


def kernel(logits, b_train_phrase, b_attitude_1, b_attitude_2, b_compare, b_matching):
    B = logits.shape[0]
    xf = logits.reshape(B, 512)
    primary = jnp.concatenate(
        [b_attitude_1, b_compare, b_train_phrase[:, -1:]], axis=1
    ).astype(jnp.int32)
    a2 = b_attitude_2.astype(jnp.int32)
    a2s = jnp.where(a2 >= _N_CLASSES, 1024, a2)
    match = b_matching.astype(jnp.int32)

    grid = (B // _TB,)
    return pl.pallas_call(
        _loss_kernel,
        grid=grid,
        in_specs=[
            pl.BlockSpec((_TB, 512), lambda i: (i, 0)),
            pl.BlockSpec((_TB, 8), lambda i: (i, 0)),
            pl.BlockSpec((_TB, 1), lambda i: (i, 0)),
            pl.BlockSpec((_TB, 1), lambda i: (i, 0)),
            pl.BlockSpec((3, 512), lambda i: (0, 0)),
            pl.BlockSpec((512, 8), lambda i: (0, 0)),
        ],
        out_specs=pl.BlockSpec((_TB, 8), lambda i: (i, 0)),
        out_shape=jax.ShapeDtypeStruct((B, 8), jnp.float32),
    )(xf, primary, a2s, match, _mm_flat(), _g_mat())


# speedup vs baseline: 1.0981x; 1.0659x over previous
import jax
import jax.numpy as jnp
import numpy as np
from jax.experimental import pallas as pl

_N_CLASSES = 64
_SELECTED_MAPS = [[3, 17, 42], [5, 9, 28, 51], [0, 12, 33], [7, 21, 44, 60], [2, 14, 39], [8, 26, 55, 63]]
_MIS_VAL = 0.5
_TB = 2048


def _mm_flat():
    m = np.zeros((8, _N_CLASSES), dtype=np.float32)
    for i, vals in enumerate(_SELECTED_MAPS):
        m[1 + i, vals] = 1.0
    mm = m.reshape(1, 512)
    mid = np.zeros((1, 512), dtype=np.float32)
    mid[0, 64:448] = 1.0
    return jnp.asarray(np.concatenate([mm, (_MIS_VAL - 1.0 - mm) * mid, mid], axis=0))


def _g_mat():
    g = np.zeros((512, 8), dtype=np.float32)
    for l in range(512):
        g[l, l >> 6] = 1.0
    return jnp.asarray(g)


def _loss_kernel(x_ref, prim_ref, a2s_ref, match_ref, mm_ref, g_ref, out_ref):
    x = x_ref[...].astype(jnp.bfloat16)
    prim = prim_ref[...]
    a2s = a2s_ref[...]
    match = match_ref[...]

    lane = jax.lax.broadcasted_iota(jnp.int32, x.shape, 1)
    b0 = (lane & 64) != 0
    b1 = (lane & 128) != 0
    b2 = (lane & 256) != 0

    def sel(col_a, col_b, bit):
        return jnp.where(bit, col_b, col_a)

    p01 = sel(prim[:, 0:1], prim[:, 1:2], b0)
    p23 = sel(prim[:, 2:3], prim[:, 3:4], b0)
    p45 = sel(prim[:, 4:5], prim[:, 5:6], b0)
    p67 = sel(prim[:, 6:7], prim[:, 7:8], b0)
    p0123 = sel(p01, p23, b1)
    p4567 = sel(p45, p67, b1)
    prim_lane = sel(p0123, p4567, b2)

    c = lane & 63
    eq_p = (c == prim_lane).astype(jnp.bfloat16)
    eq_a2 = (lane == a2s).astype(jnp.bfloat16)

    mm = mm_ref[0:1, :].astype(jnp.bfloat16)
    mmc = mm_ref[1:2, :].astype(jnp.bfloat16)
    nonmatchf = (match == 0).astype(jnp.bfloat16)
    tcoef = eq_p + eq_a2 + nonmatchf * (mm + eq_p * mmc)

    t = jnp.exp(-jnp.abs(x))
    log1p_t = ((jnp.bfloat16(0.10774686) * t - jnp.bfloat16(0.3971183)) * t
               + jnp.bfloat16(0.98239712)) * t
    sp = jnp.maximum(x, jnp.bfloat16(0.0)) + log1p_t
    y = (sp - x * tcoef).astype(jnp.float32)
    out_ref[...] = jnp.dot(y, g_ref[...], preferred_element_type=jnp.float32)


def kernel(logits, b_train_phrase, b_attitude_1, b_attitude_2, b_compare, b_matching):
    B = logits.shape[0]
    xf = logits.reshape(B, 512)
    primary = jnp.concatenate(
        [b_attitude_1, b_compare, b_train_phrase[:, -1:]], axis=1
    ).astype(jnp.int32)
    a2 = b_attitude_2.astype(jnp.int32)
    a2s = jnp.where(a2 >= _N_CLASSES, 1024, a2)
    match = b_matching.astype(jnp.int32)

    grid = (B // _TB,)
    return pl.pallas_call(
        _loss_kernel,
        grid=grid,
        in_specs=[
            pl.BlockSpec((_TB, 512), lambda i: (i, 0)),
            pl.BlockSpec((_TB, 8), lambda i: (i, 0)),
            pl.BlockSpec((_TB, 1), lambda i: (i, 0)),
            pl.BlockSpec((_TB, 1), lambda i: (i, 0)),
            pl.BlockSpec((3, 512), lambda i: (0, 0)),
            pl.BlockSpec((512, 8), lambda i: (0, 0)),
        ],
        out_specs=pl.BlockSpec((_TB, 8), lambda i: (i, 0)),
        out_shape=jax.ShapeDtypeStruct((B, 8), jnp.float32),
    )(xf, primary, a2s, match, _mm_flat(), _g_mat())


# speedup vs baseline: 1.1081x; 1.0091x over previous
import jax
import jax.numpy as jnp
import numpy as np
from jax.experimental import pallas as pl

_N_CLASSES = 64
_SELECTED_MAPS = [[3, 17, 42], [5, 9, 28, 51], [0, 12, 33], [7, 21, 44, 60], [2, 14, 39], [8, 26, 55, 63]]
_MIS_VAL = 0.5
_TB = 2048


def _mm_flat():
    m = np.zeros((8, _N_CLASSES), dtype=np.float32)
    for i, vals in enumerate(_SELECTED_MAPS):
        m[1 + i, vals] = 1.0
    mm = m.reshape(1, 512)
    mid = np.zeros((1, 512), dtype=np.float32)
    mid[0, 64:448] = 1.0
    return jnp.asarray(np.concatenate([mm, (_MIS_VAL - 1.0 - mm) * mid, mid], axis=0))


def _g_mat():
    g = np.zeros((512, 8), dtype=np.float32)
    for l in range(512):
        g[l, l >> 6] = 1.0
    return jnp.asarray(g)


def _loss_kernel(x_ref, prim_ref, a2s_ref, match_ref, mm_ref, g_ref, out_ref):
    x = x_ref[...].astype(jnp.bfloat16)
    prim = prim_ref[...]
    a2s = a2s_ref[...]
    match = match_ref[...]

    lane = jax.lax.broadcasted_iota(jnp.int32, x.shape, 1)
    b0 = (lane & 64) != 0
    b1 = (lane & 128) != 0
    b2 = (lane & 256) != 0

    def sel(col_a, col_b, bit):
        return jnp.where(bit, col_b, col_a)

    p01 = sel(prim[:, 0:1], prim[:, 1:2], b0)
    p23 = sel(prim[:, 2:3], prim[:, 3:4], b0)
    p45 = sel(prim[:, 4:5], prim[:, 5:6], b0)
    p67 = sel(prim[:, 6:7], prim[:, 7:8], b0)
    p0123 = sel(p01, p23, b1)
    p4567 = sel(p45, p67, b1)
    prim_lane = sel(p0123, p4567, b2)

    c = lane & 63
    eq_p = (c == prim_lane).astype(jnp.bfloat16)
    eq_a2 = (lane == a2s).astype(jnp.bfloat16)

    mm = mm_ref[0:1, :].astype(jnp.bfloat16)
    mmc = mm_ref[1:2, :].astype(jnp.bfloat16)
    nonmatchf = (match == 0).astype(jnp.bfloat16)
    tcoef = eq_p + eq_a2 + nonmatchf * (mm + eq_p * mmc)

    t = jnp.exp(-jnp.abs(x))
    log1p_t = ((jnp.bfloat16(0.10774686) * t - jnp.bfloat16(0.3971183)) * t
               + jnp.bfloat16(0.98239712)) * t
    sp = jnp.maximum(x, jnp.bfloat16(0.0)) + log1p_t
    y = sp - x * tcoef
    out_ref[...] = jnp.dot(y, g_ref[...].astype(jnp.bfloat16), preferred_element_type=jnp.float32)


def kernel(logits, b_train_phrase, b_attitude_1, b_attitude_2, b_compare, b_matching):
    B = logits.shape[0]
    xf = logits.reshape(B, 512)
    primary = jnp.concatenate(
        [b_attitude_1, b_compare, b_train_phrase[:, -1:]], axis=1
    ).astype(jnp.int32)
    a2 = b_attitude_2.astype(jnp.int32)
    a2s = jnp.where(a2 >= _N_CLASSES, 1024, a2)
    match = b_matching.astype(jnp.int32)

    grid = (B // _TB,)
    return pl.pallas_call(
        _loss_kernel,
        grid=grid,
        in_specs=[
            pl.BlockSpec((_TB, 512), lambda i: (i, 0)),
            pl.BlockSpec((_TB, 8), lambda i: (i, 0)),
            pl.BlockSpec((_TB, 1), lambda i: (i, 0)),
            pl.BlockSpec((_TB, 1), lambda i: (i, 0)),
            pl.BlockSpec((3, 512), lambda i: (0, 0)),
            pl.BlockSpec((512, 8), lambda i: (0, 0)),
        ],
        out_specs=pl.BlockSpec((_TB, 8), lambda i: (i, 0)),
        out_shape=jax.ShapeDtypeStruct((B, 8), jnp.float32),
    )(xf, primary, a2s, match, _mm_flat(), _g_mat())
